# baseline (speedup 1.0000x reference)
import jax
import jax.numpy as jnp
from jax import lax
from jax.experimental import pallas as pl
from jax.experimental.pallas import tpu as pltpu
from jax.experimental.pallas import tpu_sc as plsc

_N = 100000
_DIM = 32
_B, _D, _H, _W = 2, 21, 320, 320
_TOTAL = _B * _D * _H * _W
_EPS = 1e-5
_K = 27

_NTILES = 32
_PER_TILE = 3328
_NPAD = _NTILES * _PER_TILE
_GRP = 64
_NGRP = _PER_TILE // _GRP
_KP = 28

_OFFS = [((dz * _H + dy) * _W + dx, dz, dy, dx)
         for dz in (-1, 0, 1) for dy in (-1, 0, 1) for dx in (-1, 0, 1)]


def _sc_gather(table, pk, feats128):
  mesh = plsc.VectorSubcoreMesh(core_axis_name="c", subcore_axis_name="s")

  def body(tab_hbm, pk_hbm, f_hbm, g_hbm,
           pkb, zb, yb, xb, lb, qb, tb, ib, rows2, full, sem_t, sem_f):
    cid = lax.axis_index("c")
    sid = lax.axis_index("s")
    wid = sid * 2 + cid
    base = wid * _PER_TILE

    zeros16 = jnp.zeros((16,), jnp.float32)

    def z_body(u, ucarry):
      full[u, pl.ds(_K * _DIM, 16)] = zeros16
      full[u, pl.ds(_K * _DIM + 16, 16)] = zeros16
      return ucarry
    lax.fori_loop(0, _GRP, z_body, 0, unroll=8)

    def relayout(k, rbuf):
      def c_body(u, ucarry):
        full[u, pl.ds(k * _DIM, 16)] = rows2[rbuf, u, pl.ds(0, 16)]
        full[u, pl.ds(k * _DIM + 16, 16)] = rows2[rbuf, u, pl.ds(16, 16)]
        return ucarry
      lax.fori_loop(0, _GRP, c_body, 0, unroll=8)

    def grp_body(g, carry):
      v0 = base + g * _GRP
      pltpu.sync_copy(pk_hbm.at[pl.ds(v0, _GRP)], pkb)

      for u in range(_GRP // 16):
        sl = pl.ds(u * 16, 16)
        pv = pkb[sl]
        bv = (pv >> 23) & 1
        zv = (pv >> 18) & 31
        yv = (pv >> 9) & 511
        xv = pv & 511
        zb[sl] = zv
        yb[sl] = yv
        xb[sl] = xv
        lb[sl] = ((bv * _D + zv) * _H + yv) * _W + xv

      for k, (offc, dz, dy, dx) in enumerate(_OFFS):
        for u in range(_GRP // 16):
          sl = pl.ds(u * 16, 16)
          zv = zb[sl] + dz
          yv = yb[sl] + dy
          xv = xb[sl] + dx
          ok = ((zv >= 0) & (zv < _D) & (yv >= 0) & (yv < _H)
                & (xv >= 0) & (xv < _W))
          nl = lb[sl] + offc
          qb[k, sl] = jnp.where(ok, jnp.clip(nl, 0, _TOTAL - 1),
                                _TOTAL).astype(jnp.int32)

      tds = [pltpu.async_copy(tab_hbm.at[qb.at[k]], tb.at[k], sem_t)
             for k in range(_K)]
      for d in tds:
        d.wait()

      for k in range(_K):
        for u in range(_GRP // 16):
          sl = pl.ds(u * 16, 16)
          tv = tb[k, sl]
          ib[k, sl] = jnp.where(tv >= 0, tv, _N).astype(jnp.int32)

      f0 = pltpu.async_copy(f_hbm.at[ib.at[0]], rows2.at[0], sem_f)
      for k in range(_K):
        rb = k % 2
        if k + 1 < _K:
          fn = pltpu.async_copy(f_hbm.at[ib.at[k + 1]], rows2.at[1 - rb],
                                sem_f)
        f0.wait()
        relayout(k, rb)
        if k + 1 < _K:
          f0 = fn

      pltpu.sync_copy(full, g_hbm.at[pl.ds(v0, _GRP), :])
      return carry

    lax.fori_loop(0, _NGRP, grp_body, 0)

  f = pl.kernel(
      body,
      out_type=jax.ShapeDtypeStruct((_NPAD, _KP * _DIM), jnp.float32),
      mesh=mesh,
      scratch_types=[
          pltpu.VMEM((_GRP,), jnp.int32),
          pltpu.VMEM((_GRP,), jnp.int32),
          pltpu.VMEM((_GRP,), jnp.int32),
          pltpu.VMEM((_GRP,), jnp.int32),
          pltpu.VMEM((_GRP,), jnp.int32),
          pltpu.VMEM((_K, _GRP), jnp.int32),
          pltpu.VMEM((_K, _GRP), jnp.int32),
          pltpu.VMEM((_K, _GRP), jnp.int32),
          pltpu.VMEM((2, _GRP, 128), jnp.float32),
          pltpu.VMEM((_GRP, _KP * _DIM), jnp.float32),
          pltpu.SemaphoreType.DMA,
          pltpu.SemaphoreType.DMA,
      ],
  )
  return f(table, pk, feats128)


_RB = 1000


def _tc_body(g_ref, w_ref, b_ref, gam_ref, bet_ref, o_ref):
  a = g_ref[:, :]
  h = jnp.dot(a, w_ref[:, :], preferred_element_type=jnp.float32)
  h = h + b_ref[:, :]
  h = 0.5 * h * (1.0 + lax.erf(h * 0.7071067811865476))
  mu = jnp.mean(h, axis=1, keepdims=True)
  d = h - mu
  var = jnp.mean(d * d, axis=1, keepdims=True)
  o_ref[:, :] = d * lax.rsqrt(var + _EPS) * gam_ref[:, :] + bet_ref[:, :]


def _tc_conv_ln(g, wstack, bias, ln_gamma, ln_beta):
  return pl.pallas_call(
      _tc_body,
      grid=(_N // _RB,),
      in_specs=[
          pl.BlockSpec((_RB, _KP * _DIM), lambda i: (i, 0)),
          pl.BlockSpec((_KP * _DIM, _DIM), lambda i: (0, 0)),
          pl.BlockSpec((1, _DIM), lambda i: (0, 0)),
          pl.BlockSpec((1, _DIM), lambda i: (0, 0)),
          pl.BlockSpec((1, _DIM), lambda i: (0, 0)),
      ],
      out_specs=pl.BlockSpec((_RB, _DIM), lambda i: (i, 0)),
      out_shape=jax.ShapeDtypeStruct((_N, _DIM), jnp.float32),
  )(g, wstack, bias.reshape(1, _DIM), ln_gamma.reshape(1, _DIM),
    ln_beta.reshape(1, _DIM))


def kernel(features, coords, weight, bias, ln_gamma, ln_beta):
  b = coords[:, 0]
  z = coords[:, 1]
  y = coords[:, 2]
  x = coords[:, 3]
  lin = ((b * _D + z) * _H + y) * _W + x
  table = jnp.full((_TOTAL + 8,), -1, jnp.int32).at[lin].set(
      jnp.arange(_N, dtype=jnp.int32))

  pk = (b << 23) | (z << 18) | (y << 9) | x
  pk = jnp.pad(pk, (0, _NPAD - _N))
  feats128 = jnp.pad(features, ((0, _NPAD - _N), (0, 128 - _DIM)))

  g = _sc_gather(table, pk, feats128)
  wstack = jnp.pad(weight.reshape(_K * _DIM, _DIM),
                   ((0, (_KP - _K) * _DIM), (0, 0)))
  return _tc_conv_ln(g, wstack, bias, ln_gamma, ln_beta)
